# Initial kernel scaffold; baseline (speedup 1.0000x reference)
import jax
import jax.numpy as jnp
from jax.experimental import pallas as pl


def kernel(x, edge_index, W1l, b1, W1r, W2l, b2, W2r):
    raise NotImplementedError("write your pallas kernel here")


# speedup vs baseline: 4.3194x; 4.3194x over previous
import functools

import jax
import jax.numpy as jnp
from jax import lax
from jax.experimental import pallas as pl
from jax.experimental.pallas import tpu as pltpu
from jax.experimental.pallas import tpu_sc as plsc

N = 10000
D = 128
E = 320000
NC = 2
NS = 16
NW = NC * NS
C = 128
CH_PER_W = -(-E // (C * NW))
E_PAD = CH_PER_W * NW * C
NP = N + 8
NPC = NP + 8
RPT = 624
LAST_BASE = RPT * (NS - 1)
LAST_ROWS = N - LAST_BASE


def _make_sc_agg(with_count: bool):
    out_type = [jax.ShapeDtypeStruct((NC, N, D), jnp.float32)]
    scratch = [
        pltpu.VMEM((C,), jnp.int32),
        pltpu.VMEM((C,), jnp.int32),
        pltpu.VMEM((C, D), jnp.float32),
        pltpu.SemaphoreType.DMA,
        pltpu.VMEM_SHARED((NP, D), jnp.float32),
    ]
    if with_count:
        out_type.append(jax.ShapeDtypeStruct((NC, N, D), jnp.float32))

    mesh = plsc.VectorSubcoreMesh(
        core_axis_name="c", subcore_axis_name="s", num_cores=NC,
        num_subcores=NS)

    @functools.partial(pl.kernel, out_type=out_type, mesh=mesh,
                       scratch_types=scratch)
    def sc_agg(table_hbm, src_hbm, dst_hbm, *rest):
        if with_count:
            agg_out, cnt_out, srcv, dstv, rows, sem, agg_sp = rest
        else:
            agg_out, srcv, dstv, rows, sem, agg_sp = rest
        c = lax.axis_index("c")
        s = lax.axis_index("s")
        w = s * NC + c

        def fill_rows(val):
            v16 = jnp.full((16,), val, jnp.float32)

            def frow(i, _):
                for j in range(D // 16):
                    rows[i, pl.ds(j * 16, 16)] = v16
                return 0

            lax.fori_loop(0, C, frow, 0)

        fill_rows(0.0)
        base = s * RPT

        def zero_spans():
            def _zero_span(b0, sizes):
                off = 0
                for sz in sizes:
                    pltpu.sync_copy(rows.at[pl.ds(0, sz)],
                                    agg_sp.at[pl.ds(b0 + off, sz)])
                    off += sz

            @pl.when(s < NS - 1)
            def _():
                _zero_span(base, (128, 128, 128, 128, 112))

            @pl.when(s == NS - 1)
            def _():
                _zero_span(LAST_BASE, (128, 128, 128, 128, 128, 8))

        def copy_out(dst_hbm_ref):
            @pl.when(s < NS - 1)
            def _():
                pltpu.sync_copy(agg_sp.at[pl.ds(base, RPT)],
                                dst_hbm_ref.at[c, pl.ds(base, RPT)])

            @pl.when(s == NS - 1)
            def _():
                pltpu.sync_copy(agg_sp.at[pl.ds(LAST_BASE, LAST_ROWS)],
                                dst_hbm_ref.at[c, pl.ds(LAST_BASE, LAST_ROWS)])

        zero_spans()
        plsc.subcore_barrier()

        def body(i, _):
            e0 = (i * NW + w) * C
            pltpu.sync_copy(src_hbm.at[pl.ds(e0, C)], srcv)
            pltpu.sync_copy(dst_hbm.at[pl.ds(e0, C)], dstv)
            pltpu.async_copy(table_hbm.at[srcv], rows, sem).wait()
            pltpu.sync_copy(rows, agg_sp.at[dstv], add=True)
            return 0

        lax.fori_loop(0, CH_PER_W, body, 0)
        plsc.subcore_barrier()
        copy_out(agg_out)

        if with_count:
            plsc.subcore_barrier()
            fill_rows(0.0)
            zero_spans()
            fill_rows(1.0)
            plsc.subcore_barrier()

            def cbody(i, _):
                e0 = (i * NW + w) * C
                pltpu.sync_copy(dst_hbm.at[pl.ds(e0, C)], dstv)
                pltpu.sync_copy(rows, agg_sp.at[dstv], add=True)
                return 0

            lax.fori_loop(0, CH_PER_W, cbody, 0)
            plsc.subcore_barrier()
            copy_out(cnt_out)

    return sc_agg


_sc_agg_cnt = _make_sc_agg(with_count=True)
_sc_agg = _make_sc_agg(with_count=False)

_TC_BLK = 1000


def _tc_dense_body(aggp, cntp, x, wlt, wrt, b, out):
    agg = aggp[0] + aggp[1]
    cnt = cntp[0, :, 0:1] + cntp[1, :, 0:1]
    mean = agg / jnp.maximum(cnt, 1.0)
    h = (jnp.dot(mean, wlt[...], preferred_element_type=jnp.float32)
         + jnp.dot(x[...], wrt[...], preferred_element_type=jnp.float32)
         + b[...])
    out[...] = jnp.maximum(h, 0.0)


def _tc_dense(aggp, cntp, x, wlt, wrt, b):
    grid = (N // _TC_BLK,)
    return pl.pallas_call(
        _tc_dense_body,
        grid=grid,
        in_specs=[
            pl.BlockSpec((NC, _TC_BLK, D), lambda i: (0, i, 0)),
            pl.BlockSpec((NC, _TC_BLK, D), lambda i: (0, i, 0)),
            pl.BlockSpec((_TC_BLK, D), lambda i: (i, 0)),
            pl.BlockSpec((D, D), lambda i: (0, 0)),
            pl.BlockSpec((D, D), lambda i: (0, 0)),
            pl.BlockSpec((1, D), lambda i: (0, 0)),
        ],
        out_specs=pl.BlockSpec((_TC_BLK, D), lambda i: (i, 0)),
        out_shape=jax.ShapeDtypeStruct((N, D), jnp.float32),
    )(aggp, cntp, x, wlt, wrt, b)


def kernel(x, edge_index, W1l, b1, W1r, W2l, b2, W2r):
    B, n, d = x.shape
    h0 = x.reshape(N, D)
    pad = E_PAD - E
    src = jnp.concatenate([edge_index[0], jnp.zeros((pad,), jnp.int32)])
    dst = jnp.concatenate([edge_index[1], jnp.full((pad,), N, jnp.int32)])

    agg1, cnt = _sc_agg_cnt(h0, src, dst)
    h1 = _tc_dense(agg1, cnt, h0, W1l.T, W1r.T, b1.reshape(1, D))
    agg2, = _sc_agg(h1, src, dst)
    h2 = _tc_dense(agg2, cnt, h1, W2l.T, W2r.T, b2.reshape(1, D))
    return h2.reshape(B, n, D)


# speedup vs baseline: 10.6360x; 2.4624x over previous
import functools

import jax
import jax.numpy as jnp
from jax import lax
from jax.experimental import pallas as pl
from jax.experimental.pallas import tpu as pltpu
from jax.experimental.pallas import tpu_sc as plsc

N = 10000
D = 128
E = 320000
NC = 2
NS = 16
NW = NC * NS
C = 128
CH_PER_W = -(-E // (C * NW))
E_PAD = CH_PER_W * NW * C
NCHUNK = E // C
WFULL = NCHUNK - (CH_PER_W - 1) * NW
NP = N + 8
RPT = 624
LAST_BASE = RPT * (NS - 1)
LAST_ROWS = N - LAST_BASE


def _make_sc_agg(with_count: bool):
    out_type = [jax.ShapeDtypeStruct((NC, N, D), jnp.float32)]
    scratch = [
        pltpu.VMEM((C,), jnp.int32),
        pltpu.VMEM((C,), jnp.int32),
        pltpu.VMEM((CH_PER_W, C), jnp.int32),
        pltpu.VMEM((C, D), jnp.float32),
        pltpu.VMEM((C, D), jnp.float32),
        pltpu.SemaphoreType.DMA,
        pltpu.SemaphoreType.DMA,
        pltpu.VMEM_SHARED((NP, D), jnp.float32),
    ]
    if with_count:
        out_type.append(jax.ShapeDtypeStruct((NC, N, D), jnp.float32))

    mesh = plsc.VectorSubcoreMesh(
        core_axis_name="c", subcore_axis_name="s", num_cores=NC,
        num_subcores=NS)

    @functools.partial(pl.kernel, out_type=out_type, mesh=mesh,
                       scratch_types=scratch)
    def sc_agg(table_hbm, src_hbm, dst_hbm, *rest):
        if with_count:
            (agg_out, cnt_out, srcv0, srcv1, dsts, rows0, rows1, sem0, sem1,
             agg_sp) = rest
        else:
            (agg_out, srcv0, srcv1, dsts, rows0, rows1, sem0, sem1,
             agg_sp) = rest
        c = lax.axis_index("c")
        s = lax.axis_index("s")
        w = s * NC + c

        pltpu.sync_copy(dst_hbm.at[w], dsts)

        def fill_rows0(val):
            v16 = jnp.full((16,), val, jnp.float32)

            def frow(i, _):
                for j in range(D // 16):
                    rows0[i, pl.ds(j * 16, 16)] = v16
                return 0

            lax.fori_loop(0, C, frow, 0)

        base = s * RPT

        def zero_spans():
            def _zero_span(b0, sizes):
                off = 0
                for sz in sizes:
                    pltpu.sync_copy(rows0.at[pl.ds(0, sz)],
                                    agg_sp.at[pl.ds(b0 + off, sz)])
                    off += sz

            @pl.when(s < NS - 1)
            def _():
                _zero_span(base, (128, 128, 128, 128, 112))

            @pl.when(s == NS - 1)
            def _():
                _zero_span(LAST_BASE, (128, 128, 128, 128, 128, 8))

        def copy_out(dst_hbm_ref):
            @pl.when(s < NS - 1)
            def _():
                pltpu.sync_copy(agg_sp.at[pl.ds(base, RPT)],
                                dst_hbm_ref.at[c, pl.ds(base, RPT)])

            @pl.when(s == NS - 1)
            def _():
                pltpu.sync_copy(agg_sp.at[pl.ds(LAST_BASE, LAST_ROWS)],
                                dst_hbm_ref.at[c, pl.ds(LAST_BASE, LAST_ROWS)])

        fill_rows0(0.0)
        zero_spans()
        plsc.subcore_barrier()

        def gstart(i, sbuf, rbuf, sem):
            pltpu.sync_copy(src_hbm.at[pl.ds((i * NW + w) * C, C)], sbuf)
            pltpu.async_copy(table_hbm.at[sbuf], rbuf, sem)

        def gwait(sbuf, rbuf, sem):
            pltpu.make_async_copy(table_hbm.at[sbuf], rbuf, sem).wait()

        def scat(i, rbuf):
            pltpu.sync_copy(rbuf, agg_sp.at[dsts.at[i]], add=True)

        gstart(0, srcv0, rows0, sem0)

        def pair(p, _):
            i = 2 * p
            gstart(i + 1, srcv1, rows1, sem1)
            gwait(srcv0, rows0, sem0)
            scat(i, rows0)
            gstart(i + 2, srcv0, rows0, sem0)
            gwait(srcv1, rows1, sem1)
            scat(i + 1, rows1)
            return 0

        lax.fori_loop(0, (CH_PER_W - 1) // 2, pair, 0)
        gwait(srcv0, rows0, sem0)

        @pl.when(w < WFULL)
        def _():
            scat(CH_PER_W - 1, rows0)

        plsc.subcore_barrier()
        copy_out(agg_out)

        if with_count:
            plsc.subcore_barrier()
            fill_rows0(0.0)
            zero_spans()
            fill_rows0(1.0)
            plsc.subcore_barrier()
            nch = CH_PER_W - 1 + jnp.where(w < WFULL, 1, 0)

            def cfire(i, _):
                pltpu.async_copy(rows0, agg_sp.at[dsts.at[i]], sem0,
                                 add=True)
                return 0

            def cdrain(i, _):
                pltpu.make_async_copy(rows0, agg_sp.at[dsts.at[0]],
                                      sem0).wait()
                return 0

            lax.fori_loop(0, nch, cfire, 0)
            lax.fori_loop(0, nch, cdrain, 0)
            plsc.subcore_barrier()
            copy_out(cnt_out)

    return sc_agg


_sc_agg_cnt = _make_sc_agg(with_count=True)
_sc_agg = _make_sc_agg(with_count=False)

_TC_BLK = 1000


def _tc_dense_body(aggp, cntp, x, wlt, wrt, b, out):
    agg = aggp[0] + aggp[1]
    cnt = cntp[0, :, 0:1] + cntp[1, :, 0:1]
    mean = agg / jnp.maximum(cnt, 1.0)
    h = (jnp.dot(mean, wlt[...], preferred_element_type=jnp.float32)
         + jnp.dot(x[...], wrt[...], preferred_element_type=jnp.float32)
         + b[...])
    out[...] = jnp.maximum(h, 0.0)


def _tc_dense(aggp, cntp, x, wlt, wrt, b):
    grid = (N // _TC_BLK,)
    return pl.pallas_call(
        _tc_dense_body,
        grid=grid,
        in_specs=[
            pl.BlockSpec((NC, _TC_BLK, D), lambda i: (0, i, 0)),
            pl.BlockSpec((NC, _TC_BLK, D), lambda i: (0, i, 0)),
            pl.BlockSpec((_TC_BLK, D), lambda i: (i, 0)),
            pl.BlockSpec((D, D), lambda i: (0, 0)),
            pl.BlockSpec((D, D), lambda i: (0, 0)),
            pl.BlockSpec((1, D), lambda i: (0, 0)),
        ],
        out_specs=pl.BlockSpec((_TC_BLK, D), lambda i: (i, 0)),
        out_shape=jax.ShapeDtypeStruct((N, D), jnp.float32),
    )(aggp, cntp, x, wlt, wrt, b)


def kernel(x, edge_index, W1l, b1, W1r, W2l, b2, W2r):
    B, n, d = x.shape
    h0 = x.reshape(N, D)
    pad = E_PAD - E
    srcp = jnp.concatenate(
        [edge_index[0], (jnp.arange(pad, dtype=jnp.int32) * 29) % N])
    dstp = jnp.concatenate([edge_index[1], jnp.zeros((pad,), jnp.int32)])
    dst_w = dstp.reshape(CH_PER_W, NW, C).transpose(1, 0, 2)

    agg1, cnt = _sc_agg_cnt(h0, srcp, dst_w)
    h1 = _tc_dense(agg1, cnt, h0, W1l.T, W1r.T, b1.reshape(1, D))
    agg2, = _sc_agg(h1, srcp, dst_w)
    h2 = _tc_dense(agg2, cnt, h1, W2l.T, W2r.T, b2.reshape(1, D))
    return h2.reshape(B, n, D)


# speedup vs baseline: 11.5277x; 1.0838x over previous
import functools

import jax
import jax.numpy as jnp
from jax import lax
from jax.experimental import pallas as pl
from jax.experimental.pallas import tpu as pltpu
from jax.experimental.pallas import tpu_sc as plsc

N = 10000
D = 128
E = 320000
NC = 2
NS = 16
NW = NC * NS
C = 128
CH_PER_W = -(-E // (C * NW))
E_PAD = CH_PER_W * NW * C
NCHUNK = E // C
WFULL = NCHUNK - (CH_PER_W - 1) * NW
NP = N + 8
RPT = 624
LAST_BASE = RPT * (NS - 1)
LAST_ROWS = N - LAST_BASE


def _make_sc_agg(with_count: bool):
    out_type = [jax.ShapeDtypeStruct((NC, N, D), jnp.float32)]
    scratch = [
        pltpu.VMEM((C,), jnp.int32),
        pltpu.VMEM((C,), jnp.int32),
        pltpu.VMEM((CH_PER_W, C), jnp.int32),
        pltpu.VMEM((C, D), jnp.float32),
        pltpu.VMEM((C, D), jnp.float32),
        pltpu.SemaphoreType.DMA,
        pltpu.SemaphoreType.DMA,
        pltpu.SemaphoreType.DMA,
        pltpu.SemaphoreType.DMA,
        pltpu.VMEM_SHARED((NP, D), jnp.float32),
    ]
    if with_count:
        out_type.append(jax.ShapeDtypeStruct((NC, N, D), jnp.float32))

    mesh = plsc.VectorSubcoreMesh(
        core_axis_name="c", subcore_axis_name="s", num_cores=NC,
        num_subcores=NS)

    @functools.partial(pl.kernel, out_type=out_type, mesh=mesh,
                       scratch_types=scratch)
    def sc_agg(table_hbm, src_hbm, dst_hbm, *rest):
        if with_count:
            (agg_out, cnt_out, srcv0, srcv1, dsts, rows0, rows1, sem0, sem1,
             ssem0, ssem1, agg_sp) = rest
        else:
            (agg_out, srcv0, srcv1, dsts, rows0, rows1, sem0, sem1,
             ssem0, ssem1, agg_sp) = rest
        c = lax.axis_index("c")
        s = lax.axis_index("s")
        w = s * NC + c

        pltpu.sync_copy(dst_hbm.at[w], dsts)

        def fill_rows0(val):
            v16 = jnp.full((16,), val, jnp.float32)

            def frow(i, _):
                for j in range(D // 16):
                    rows0[i, pl.ds(j * 16, 16)] = v16
                return 0

            lax.fori_loop(0, C, frow, 0)

        base = s * RPT

        def zero_spans():
            def _zero_span(b0, sizes):
                off = 0
                for sz in sizes:
                    pltpu.sync_copy(rows0.at[pl.ds(0, sz)],
                                    agg_sp.at[pl.ds(b0 + off, sz)])
                    off += sz

            @pl.when(s < NS - 1)
            def _():
                _zero_span(base, (128, 128, 128, 128, 112))

            @pl.when(s == NS - 1)
            def _():
                _zero_span(LAST_BASE, (128, 128, 128, 128, 128, 8))

        def copy_out(dst_hbm_ref):
            @pl.when(s < NS - 1)
            def _():
                pltpu.sync_copy(agg_sp.at[pl.ds(base, RPT)],
                                dst_hbm_ref.at[c, pl.ds(base, RPT)])

            @pl.when(s == NS - 1)
            def _():
                pltpu.sync_copy(agg_sp.at[pl.ds(LAST_BASE, LAST_ROWS)],
                                dst_hbm_ref.at[c, pl.ds(LAST_BASE, LAST_ROWS)])

        fill_rows0(0.0)
        zero_spans()
        plsc.subcore_barrier()

        def sload(i, sbuf, ssem):
            pltpu.async_copy(src_hbm.at[pl.ds((i * NW + w) * C, C)], sbuf,
                             ssem)

        def swait(sbuf, ssem):
            pltpu.make_async_copy(src_hbm.at[pl.ds(0, C)], sbuf, ssem).wait()

        def gstart(sbuf, rbuf, sem):
            pltpu.async_copy(table_hbm.at[sbuf], rbuf, sem)

        def gwait(sbuf, rbuf, sem):
            pltpu.make_async_copy(table_hbm.at[sbuf], rbuf, sem).wait()

        def scat(i, rbuf):
            pltpu.sync_copy(rbuf, agg_sp.at[dsts.at[i]], add=True)

        pltpu.sync_copy(src_hbm.at[pl.ds(w * C, C)], srcv0)
        gstart(srcv0, rows0, sem0)
        sload(1, srcv1, ssem1)
        npairs = (CH_PER_W - 1) // 2

        def pair(p, _):
            i = 2 * p
            swait(srcv1, ssem1)
            gstart(srcv1, rows1, sem1)
            gwait(srcv0, rows0, sem0)
            sload(i + 2, srcv0, ssem0)
            scat(i, rows0)
            swait(srcv0, ssem0)
            gstart(srcv0, rows0, sem0)
            gwait(srcv1, rows1, sem1)

            @pl.when(p < npairs - 1)
            def _():
                sload(i + 3, srcv1, ssem1)

            scat(i + 1, rows1)
            return 0

        lax.fori_loop(0, npairs, pair, 0)
        gwait(srcv0, rows0, sem0)

        @pl.when(w < WFULL)
        def _():
            scat(CH_PER_W - 1, rows0)

        plsc.subcore_barrier()
        copy_out(agg_out)

        if with_count:
            plsc.subcore_barrier()
            fill_rows0(0.0)
            zero_spans()
            fill_rows0(1.0)
            plsc.subcore_barrier()
            nch = CH_PER_W - 1 + jnp.where(w < WFULL, 1, 0)

            def cfire(i, _):
                pltpu.async_copy(rows0, agg_sp.at[dsts.at[i]], sem0,
                                 add=True)
                return 0

            def cdrain(i, _):
                pltpu.make_async_copy(rows0, agg_sp.at[dsts.at[0]],
                                      sem0).wait()
                return 0

            lax.fori_loop(0, nch, cfire, 0)
            lax.fori_loop(0, nch, cdrain, 0)
            plsc.subcore_barrier()
            copy_out(cnt_out)

    return sc_agg


_sc_agg_cnt = _make_sc_agg(with_count=True)
_sc_agg = _make_sc_agg(with_count=False)

_TC_BLK = 1000


def _tc_dense_body(aggp, cntp, x, wlt, wrt, b, out):
    agg = aggp[0] + aggp[1]
    cnt = cntp[0, :, 0:1] + cntp[1, :, 0:1]
    mean = agg / jnp.maximum(cnt, 1.0)
    h = (jnp.dot(mean, wlt[...], preferred_element_type=jnp.float32)
         + jnp.dot(x[...], wrt[...], preferred_element_type=jnp.float32)
         + b[...])
    out[...] = jnp.maximum(h, 0.0)


def _tc_dense(aggp, cntp, x, wlt, wrt, b):
    grid = (N // _TC_BLK,)
    return pl.pallas_call(
        _tc_dense_body,
        grid=grid,
        in_specs=[
            pl.BlockSpec((NC, _TC_BLK, D), lambda i: (0, i, 0)),
            pl.BlockSpec((NC, _TC_BLK, D), lambda i: (0, i, 0)),
            pl.BlockSpec((_TC_BLK, D), lambda i: (i, 0)),
            pl.BlockSpec((D, D), lambda i: (0, 0)),
            pl.BlockSpec((D, D), lambda i: (0, 0)),
            pl.BlockSpec((1, D), lambda i: (0, 0)),
        ],
        out_specs=pl.BlockSpec((_TC_BLK, D), lambda i: (i, 0)),
        out_shape=jax.ShapeDtypeStruct((N, D), jnp.float32),
    )(aggp, cntp, x, wlt, wrt, b)


def kernel(x, edge_index, W1l, b1, W1r, W2l, b2, W2r):
    B, n, d = x.shape
    h0 = x.reshape(N, D)
    pad = E_PAD - E
    srcp = jnp.concatenate(
        [edge_index[0], (jnp.arange(pad, dtype=jnp.int32) * 29) % N])
    dstp = jnp.concatenate([edge_index[1], jnp.zeros((pad,), jnp.int32)])
    dst_w = dstp.reshape(CH_PER_W, NW, C).transpose(1, 0, 2)

    agg1, cnt = _sc_agg_cnt(h0, srcp, dst_w)
    h1 = _tc_dense(agg1, cnt, h0, W1l.T, W1r.T, b1.reshape(1, D))
    agg2, = _sc_agg(h1, srcp, dst_w)
    h2 = _tc_dense(agg2, cnt, h1, W2l.T, W2r.T, b2.reshape(1, D))
    return h2.reshape(B, n, D)


# speedup vs baseline: 11.5732x; 1.0039x over previous
import functools

import jax
import jax.numpy as jnp
from jax import lax
from jax.experimental import pallas as pl
from jax.experimental.pallas import tpu as pltpu
from jax.experimental.pallas import tpu_sc as plsc

N = 10000
D = 128
E = 320000
NC = 2
NS = 16
NW = NC * NS
C = 128
CH_PER_W = -(-E // (C * NW))
E_PAD = CH_PER_W * NW * C
NCHUNK = E // C
WFULL = NCHUNK - (CH_PER_W - 1) * NW
NP = N
RPT = 624
LAST_BASE = RPT * (NS - 1)
LAST_ROWS = N - LAST_BASE


def _make_sc_agg(with_count: bool):
    out_type = [jax.ShapeDtypeStruct((NC, N, D), jnp.float32)]
    scratch = [
        pltpu.VMEM((C,), jnp.int32),
        pltpu.VMEM((C,), jnp.int32),
        pltpu.VMEM((CH_PER_W, C), jnp.int32),
        pltpu.VMEM((C, D), jnp.float32),
        pltpu.VMEM((C, D), jnp.float32),
        pltpu.SemaphoreType.DMA,
        pltpu.SemaphoreType.DMA,
        pltpu.SemaphoreType.DMA,
        pltpu.SemaphoreType.DMA,
        pltpu.VMEM_SHARED((NP, D), jnp.float32),
    ]
    if with_count:
        out_type.append(jax.ShapeDtypeStruct((NC, N, D), jnp.float32))

    mesh = plsc.VectorSubcoreMesh(
        core_axis_name="c", subcore_axis_name="s", num_cores=NC,
        num_subcores=NS)

    @functools.partial(pl.kernel, out_type=out_type, mesh=mesh,
                       scratch_types=scratch)
    def sc_agg(table_hbm, src_hbm, dst_hbm, *rest):
        if with_count:
            (agg_out, cnt_out, srcv0, srcv1, dsts, rows0, rows1, sem0, sem1,
             ssem0, ssem1, agg_sp) = rest
        else:
            (agg_out, srcv0, srcv1, dsts, rows0, rows1, sem0, sem1,
             ssem0, ssem1, agg_sp) = rest
        c = lax.axis_index("c")
        s = lax.axis_index("s")
        w = s * NC + c

        pltpu.sync_copy(dst_hbm.at[w], dsts)

        def fill_rows0(val):
            v16 = jnp.full((16,), val, jnp.float32)

            def frow(i, _):
                for j in range(D // 16):
                    rows0[i, pl.ds(j * 16, 16)] = v16
                return 0

            lax.fori_loop(0, C, frow, 0)

        base = s * RPT

        def zero_spans():
            def _zero_span(b0, sizes):
                off = 0
                for sz in sizes:
                    pltpu.async_copy(rows0.at[pl.ds(0, sz)],
                                     agg_sp.at[pl.ds(b0 + off, sz)], sem0)
                    off += sz
                for sz in sizes:
                    pltpu.make_async_copy(
                        rows0.at[pl.ds(0, sz)],
                        agg_sp.at[pl.ds(b0, sz)], sem0).wait()

            @pl.when(s < NS - 1)
            def _():
                _zero_span(base, (128, 128, 128, 128, 112))

            @pl.when(s == NS - 1)
            def _():
                _zero_span(LAST_BASE, (128, 128, 128, 128, 128))

        def copy_out(dst_hbm_ref, ncols=None):
            def one(b0, nrows):
                if ncols is None:
                    pltpu.sync_copy(agg_sp.at[pl.ds(b0, nrows)],
                                    dst_hbm_ref.at[c, pl.ds(b0, nrows)])
                else:
                    pltpu.sync_copy(
                        agg_sp.at[pl.ds(b0, nrows), pl.ds(0, ncols)],
                        dst_hbm_ref.at[c, pl.ds(b0, nrows)])

            @pl.when(s < NS - 1)
            def _():
                one(base, RPT)

            @pl.when(s == NS - 1)
            def _():
                one(LAST_BASE, LAST_ROWS)

        fill_rows0(0.0)
        zero_spans()
        plsc.subcore_barrier()

        def sload(i, sbuf, ssem):
            pltpu.async_copy(src_hbm.at[pl.ds((i * NW + w) * C, C)], sbuf,
                             ssem)

        def swait(sbuf, ssem):
            pltpu.make_async_copy(src_hbm.at[pl.ds(0, C)], sbuf, ssem).wait()

        def gstart(sbuf, rbuf, sem):
            pltpu.async_copy(table_hbm.at[sbuf], rbuf, sem)

        def gwait(sbuf, rbuf, sem):
            pltpu.make_async_copy(table_hbm.at[sbuf], rbuf, sem).wait()

        def scat(i, rbuf):
            pltpu.sync_copy(rbuf, agg_sp.at[dsts.at[i]], add=True)

        pltpu.sync_copy(src_hbm.at[pl.ds(w * C, C)], srcv0)
        gstart(srcv0, rows0, sem0)
        sload(1, srcv1, ssem1)
        npairs = (CH_PER_W - 1) // 2

        def pair(p, _):
            i = 2 * p
            swait(srcv1, ssem1)
            gstart(srcv1, rows1, sem1)
            gwait(srcv0, rows0, sem0)
            sload(i + 2, srcv0, ssem0)
            scat(i, rows0)
            swait(srcv0, ssem0)
            gstart(srcv0, rows0, sem0)
            gwait(srcv1, rows1, sem1)

            @pl.when(p < npairs - 1)
            def _():
                sload(i + 3, srcv1, ssem1)

            scat(i + 1, rows1)
            return 0

        lax.fori_loop(0, npairs, pair, 0)
        gwait(srcv0, rows0, sem0)

        @pl.when(w < WFULL)
        def _():
            scat(CH_PER_W - 1, rows0)

        plsc.subcore_barrier()
        copy_out(agg_out)

        if with_count:
            plsc.subcore_barrier()
            fill_rows0(0.0)
            zero_spans()
            fill_rows0(1.0)
            plsc.subcore_barrier()
            nch = CH_PER_W - 1 + jnp.where(w < WFULL, 1, 0)

            def cfire(i, _):
                pltpu.async_copy(rows0, agg_sp.at[dsts.at[i]], sem0,
                                 add=True)
                return 0

            def cdrain(i, _):
                pltpu.make_async_copy(rows0, agg_sp.at[dsts.at[0]],
                                      sem0).wait()
                return 0

            lax.fori_loop(0, nch, cfire, 0)
            lax.fori_loop(0, nch, cdrain, 0)
            plsc.subcore_barrier()
            copy_out(cnt_out)

    return sc_agg


_sc_agg_cnt = _make_sc_agg(with_count=True)
_sc_agg = _make_sc_agg(with_count=False)

_TC_BLK = 1000


def _tc_dense_body(aggp, cntp, x, wlt, wrt, b, out):
    agg = aggp[0] + aggp[1]
    cnt = cntp[0, :, 0:1] + cntp[1, :, 0:1]
    mean = agg / jnp.maximum(cnt, 1.0)
    h = (jnp.dot(mean, wlt[...], preferred_element_type=jnp.float32)
         + jnp.dot(x[...], wrt[...], preferred_element_type=jnp.float32)
         + b[...])
    out[...] = jnp.maximum(h, 0.0)


def _tc_dense(aggp, cntp, x, wlt, wrt, b):
    grid = (N // _TC_BLK,)
    return pl.pallas_call(
        _tc_dense_body,
        grid=grid,
        in_specs=[
            pl.BlockSpec((NC, _TC_BLK, D), lambda i: (0, i, 0)),
            pl.BlockSpec((NC, _TC_BLK, 16), lambda i: (0, i, 0)),
            pl.BlockSpec((_TC_BLK, D), lambda i: (i, 0)),
            pl.BlockSpec((D, D), lambda i: (0, 0)),
            pl.BlockSpec((D, D), lambda i: (0, 0)),
            pl.BlockSpec((1, D), lambda i: (0, 0)),
        ],
        out_specs=pl.BlockSpec((_TC_BLK, D), lambda i: (i, 0)),
        out_shape=jax.ShapeDtypeStruct((N, D), jnp.float32),
    )(aggp, cntp, x, wlt, wrt, b)


def kernel(x, edge_index, W1l, b1, W1r, W2l, b2, W2r):
    B, n, d = x.shape
    h0 = x.reshape(N, D)
    pad = E_PAD - E
    srcp = jnp.concatenate(
        [edge_index[0], (jnp.arange(pad, dtype=jnp.int32) * 29) % N])
    dstp = jnp.concatenate([edge_index[1], jnp.zeros((pad,), jnp.int32)])
    dst_w = dstp.reshape(CH_PER_W, NW, C).transpose(1, 0, 2)

    agg1, cnt = _sc_agg_cnt(h0, srcp, dst_w)
    cnt = lax.slice(cnt, (0, 0, 0), (NC, N, 16))
    h1 = _tc_dense(agg1, cnt, h0, W1l.T, W1r.T, b1.reshape(1, D))
    agg2, = _sc_agg(h1, srcp, dst_w)
    h2 = _tc_dense(agg2, cnt, h1, W2l.T, W2r.T, b2.reshape(1, D))
    return h2.reshape(B, n, D)


# speedup vs baseline: 11.5878x; 1.0013x over previous
import functools

import jax
import jax.numpy as jnp
from jax import lax
from jax.experimental import pallas as pl
from jax.experimental.pallas import tpu as pltpu
from jax.experimental.pallas import tpu_sc as plsc

N = 10000
D = 128
E = 320000
NC = 2
NS = 16
NW = NC * NS
C = 128
CH_PER_W = -(-E // (C * NW))
E_PAD = CH_PER_W * NW * C
NCHUNK = E // C
WFULL = NCHUNK - (CH_PER_W - 1) * NW
NP = N
RPT = 624
LAST_BASE = RPT * (NS - 1)
LAST_ROWS = N - LAST_BASE


def _make_sc_agg(with_count: bool):
    out_type = [jax.ShapeDtypeStruct((NC, N, D), jnp.float32)]
    scratch = [
        pltpu.VMEM((C,), jnp.int32),
        pltpu.VMEM((C,), jnp.int32),
        pltpu.VMEM((CH_PER_W, C), jnp.int32),
        pltpu.VMEM((C, D), jnp.float32),
        pltpu.VMEM((C, D), jnp.float32),
        pltpu.SemaphoreType.DMA,
        pltpu.SemaphoreType.DMA,
        pltpu.SemaphoreType.DMA,
        pltpu.SemaphoreType.DMA,
        pltpu.VMEM_SHARED((NP, D), jnp.float32),
    ]
    if with_count:
        out_type.append(jax.ShapeDtypeStruct((NC, N, D), jnp.float32))

    mesh = plsc.VectorSubcoreMesh(
        core_axis_name="c", subcore_axis_name="s", num_cores=NC,
        num_subcores=NS)

    @functools.partial(pl.kernel, out_type=out_type, mesh=mesh,
                       scratch_types=scratch)
    def sc_agg(table_hbm, src_hbm, dst_hbm, *rest):
        if with_count:
            (agg_out, cnt_out, srcv0, srcv1, dsts, rows0, rows1, sem0, sem1,
             ssem0, ssem1, agg_sp) = rest
        else:
            (agg_out, srcv0, srcv1, dsts, rows0, rows1, sem0, sem1,
             ssem0, ssem1, agg_sp) = rest
        c = lax.axis_index("c")
        s = lax.axis_index("s")
        w = s * NC + c

        pltpu.sync_copy(dst_hbm.at[:, w], dsts)

        def fill_rows0(val):
            v16 = jnp.full((16,), val, jnp.float32)

            def frow(i, _):
                for j in range(D // 16):
                    rows0[i, pl.ds(j * 16, 16)] = v16
                return 0

            lax.fori_loop(0, C, frow, 0)

        base = s * RPT

        def zero_spans():
            def _zero_span(b0, sizes):
                off = 0
                for sz in sizes:
                    pltpu.async_copy(rows0.at[pl.ds(0, sz)],
                                     agg_sp.at[pl.ds(b0 + off, sz)], sem0)
                    off += sz
                for sz in sizes:
                    pltpu.make_async_copy(
                        rows0.at[pl.ds(0, sz)],
                        agg_sp.at[pl.ds(b0, sz)], sem0).wait()

            @pl.when(s < NS - 1)
            def _():
                _zero_span(base, (128, 128, 128, 128, 112))

            @pl.when(s == NS - 1)
            def _():
                _zero_span(LAST_BASE, (128, 128, 128, 128, 128))

        def copy_out(dst_hbm_ref, ncols=None):
            def one(b0, nrows):
                if ncols is None:
                    pltpu.sync_copy(agg_sp.at[pl.ds(b0, nrows)],
                                    dst_hbm_ref.at[c, pl.ds(b0, nrows)])
                else:
                    pltpu.sync_copy(
                        agg_sp.at[pl.ds(b0, nrows), pl.ds(0, ncols)],
                        dst_hbm_ref.at[c, pl.ds(b0, nrows)])

            @pl.when(s < NS - 1)
            def _():
                one(base, RPT)

            @pl.when(s == NS - 1)
            def _():
                one(LAST_BASE, LAST_ROWS)

        fill_rows0(0.0)
        zero_spans()
        plsc.subcore_barrier()

        def sload(i, sbuf, ssem):
            pltpu.async_copy(src_hbm.at[pl.ds((i * NW + w) * C, C)], sbuf,
                             ssem)

        def swait(sbuf, ssem):
            pltpu.make_async_copy(src_hbm.at[pl.ds(0, C)], sbuf, ssem).wait()

        def gstart(sbuf, rbuf, sem):
            pltpu.async_copy(table_hbm.at[sbuf], rbuf, sem)

        def gwait(sbuf, rbuf, sem):
            pltpu.make_async_copy(table_hbm.at[sbuf], rbuf, sem).wait()

        def scat(i, rbuf):
            pltpu.sync_copy(rbuf, agg_sp.at[dsts.at[i]], add=True)

        pltpu.sync_copy(src_hbm.at[pl.ds(w * C, C)], srcv0)
        gstart(srcv0, rows0, sem0)
        sload(1, srcv1, ssem1)
        npairs = (CH_PER_W - 1) // 2

        def pair(p, _):
            i = 2 * p
            swait(srcv1, ssem1)
            gstart(srcv1, rows1, sem1)
            gwait(srcv0, rows0, sem0)
            sload(i + 2, srcv0, ssem0)
            scat(i, rows0)
            swait(srcv0, ssem0)
            gstart(srcv0, rows0, sem0)
            gwait(srcv1, rows1, sem1)

            @pl.when(p < npairs - 1)
            def _():
                sload(i + 3, srcv1, ssem1)

            scat(i + 1, rows1)
            return 0

        lax.fori_loop(0, npairs, pair, 0)
        gwait(srcv0, rows0, sem0)

        @pl.when(w < WFULL)
        def _():
            scat(CH_PER_W - 1, rows0)

        plsc.subcore_barrier()
        copy_out(agg_out)

        if with_count:
            plsc.subcore_barrier()
            fill_rows0(0.0)
            zero_spans()
            fill_rows0(1.0)
            plsc.subcore_barrier()
            nch = CH_PER_W - 1 + jnp.where(w < WFULL, 1, 0)

            def cfire(i, _):
                pltpu.async_copy(rows0, agg_sp.at[dsts.at[i]], sem0,
                                 add=True)
                return 0

            def cdrain(i, _):
                pltpu.make_async_copy(rows0, agg_sp.at[dsts.at[0]],
                                      sem0).wait()
                return 0

            lax.fori_loop(0, nch, cfire, 0)
            lax.fori_loop(0, nch, cdrain, 0)
            plsc.subcore_barrier()
            copy_out(cnt_out)

    return sc_agg


_sc_agg_cnt = _make_sc_agg(with_count=True)
_sc_agg = _make_sc_agg(with_count=False)

_TC_BLK = 1000


def _tc_dense_body(aggp, cntp, x, wlt, wrt, b, out):
    agg = aggp[0] + aggp[1]
    cnt = cntp[0, :, 0:1] + cntp[1, :, 0:1]
    mean = agg / jnp.maximum(cnt, 1.0)
    h = (jnp.dot(mean, wlt[...], preferred_element_type=jnp.float32)
         + jnp.dot(x[...], wrt[...], preferred_element_type=jnp.float32)
         + b[...])
    out[...] = jnp.maximum(h, 0.0)


def _tc_dense(aggp, cntp, x, wlt, wrt, b):
    grid = (N // _TC_BLK,)
    return pl.pallas_call(
        _tc_dense_body,
        grid=grid,
        in_specs=[
            pl.BlockSpec((NC, _TC_BLK, D), lambda i: (0, i, 0)),
            pl.BlockSpec((NC, _TC_BLK, 16), lambda i: (0, i, 0)),
            pl.BlockSpec((_TC_BLK, D), lambda i: (i, 0)),
            pl.BlockSpec((D, D), lambda i: (0, 0)),
            pl.BlockSpec((D, D), lambda i: (0, 0)),
            pl.BlockSpec((1, D), lambda i: (0, 0)),
        ],
        out_specs=pl.BlockSpec((_TC_BLK, D), lambda i: (i, 0)),
        out_shape=jax.ShapeDtypeStruct((N, D), jnp.float32),
    )(aggp, cntp, x, wlt, wrt, b)


def kernel(x, edge_index, W1l, b1, W1r, W2l, b2, W2r):
    B, n, d = x.shape
    h0 = x.reshape(N, D)
    pad = E_PAD - E
    srcp = jnp.concatenate(
        [edge_index[0], (jnp.arange(pad, dtype=jnp.int32) * 29) % N])
    dstp = jnp.concatenate([edge_index[1], jnp.zeros((pad,), jnp.int32)])
    dst_w = dstp.reshape(CH_PER_W, NW, C)

    agg1, cnt = _sc_agg_cnt(h0, srcp, dst_w)
    cnt = lax.slice(cnt, (0, 0, 0), (NC, N, 16))
    h1 = _tc_dense(agg1, cnt, h0, W1l.T, W1r.T, b1.reshape(1, D))
    agg2, = _sc_agg(h1, srcp, dst_w)
    h2 = _tc_dense(agg2, cnt, h1, W2l.T, W2r.T, b2.reshape(1, D))
    return h2.reshape(B, n, D)
